# initial kernel scaffold (unmeasured)
import jax
import jax.numpy as jnp
from jax import lax
from jax.experimental import pallas as pl
from jax.experimental.pallas import tpu as pltpu


def kernel(
    t,
):
    def body(*refs):
        pass

    out_shape = jax.ShapeDtypeStruct(..., jnp.float32)
    return pl.pallas_call(body, out_shape=out_shape)(...)


# baseline (device time: 28768 ns/iter reference)
import functools

import jax
import jax.numpy as jnp
from jax import lax
from jax.experimental import pallas as pl
from jax.experimental.pallas import tpu as pltpu

MASKS = (1, 3, 4, 8)
N_STAGES = len(MASKS)


def kernel(t):
    m, n = t.shape

    def body(x_ref, out_ref, send_buf, recv_bufs, send_sem, recv_sems):
        my_i = lax.axis_index("i")

        barrier_sem = pltpu.get_barrier_semaphore()
        for mask in MASKS:
            pl.semaphore_signal(
                barrier_sem, inc=1,
                device_id=(my_i ^ mask,),
                device_id_type=pl.DeviceIdType.MESH,
            )
        pl.semaphore_wait(barrier_sem, N_STAGES)

        out_ref[:, :] = x_ref[:, :]

        for k, mask in enumerate(MASKS):
            partner = my_i ^ mask
            send_buf[:, :] = out_ref[:, :].astype(jnp.bfloat16)
            rdma = pltpu.make_async_remote_copy(
                src_ref=send_buf,
                dst_ref=recv_bufs.at[k],
                send_sem=send_sem,
                recv_sem=recv_sems.at[k],
                device_id=(partner,),
                device_id_type=pl.DeviceIdType.MESH,
            )
            rdma.start()
            rdma.wait()
            out_ref[:, :] = out_ref[:, :] + recv_bufs[k].astype(jnp.float32)

        s = out_ref[:, :]
        r = jnp.maximum(s, 0.0)
        out_ref[:, :] = jnp.tanh(s) * s * s + r * r * r

        @functools.partial(
            pl.run_scoped, exit_barrier=pltpu.SemaphoreType.REGULAR
        )
        def _(exit_barrier):
            for mask in MASKS:
                pl.semaphore_signal(
                    exit_barrier, inc=1,
                    device_id=(my_i ^ mask,),
                    device_id_type=pl.DeviceIdType.MESH,
                )
            pl.semaphore_wait(exit_barrier, N_STAGES)

    return pl.pallas_call(
        body,
        out_shape=jax.ShapeDtypeStruct((m, n), jnp.float32),
        in_specs=[pl.BlockSpec(memory_space=pltpu.VMEM)],
        out_specs=pl.BlockSpec(memory_space=pltpu.VMEM),
        scratch_shapes=[
            pltpu.VMEM((m, n), jnp.bfloat16),
            pltpu.VMEM((N_STAGES, m, n), jnp.bfloat16),
            pltpu.SemaphoreType.DMA,
            pltpu.SemaphoreType.DMA((N_STAGES,)),
        ],
        compiler_params=pltpu.CompilerParams(collective_id=0),
    )(t)


# device time: 28565 ns/iter; 1.0071x vs baseline; 1.0071x over previous
import functools

import jax
import jax.numpy as jnp
from jax import lax
from jax.experimental import pallas as pl
from jax.experimental.pallas import tpu as pltpu

MASKS = (1, 3, 4, 8)
N_STAGES = len(MASKS)


def kernel(t):
    m, n = t.shape

    def body(x_ref, out_ref, acc, recv_bufs, send_sem, recv_sems):
        my_i = lax.axis_index("i")

        barrier_sem = pltpu.get_barrier_semaphore()
        for mask in MASKS:
            pl.semaphore_signal(
                barrier_sem, inc=1,
                device_id=(my_i ^ mask,),
                device_id_type=pl.DeviceIdType.MESH,
            )
        pl.semaphore_wait(barrier_sem, N_STAGES)

        acc[:, :] = x_ref[:, :].astype(jnp.bfloat16)

        for k, mask in enumerate(MASKS):
            partner = my_i ^ mask
            rdma = pltpu.make_async_remote_copy(
                src_ref=acc,
                dst_ref=recv_bufs.at[k],
                send_sem=send_sem,
                recv_sem=recv_sems.at[k],
                device_id=(partner,),
                device_id_type=pl.DeviceIdType.MESH,
            )
            rdma.start()
            rdma.wait()
            acc[:, :] = acc[:, :] + recv_bufs[k]

        s = acc[:, :].astype(jnp.float32)
        r = jnp.maximum(s, 0.0)
        out_ref[:, :] = jnp.tanh(s) * s * s + r * r * r

        @functools.partial(
            pl.run_scoped, exit_barrier=pltpu.SemaphoreType.REGULAR
        )
        def _(exit_barrier):
            for mask in MASKS:
                pl.semaphore_signal(
                    exit_barrier, inc=1,
                    device_id=(my_i ^ mask,),
                    device_id_type=pl.DeviceIdType.MESH,
                )
            pl.semaphore_wait(exit_barrier, N_STAGES)

    return pl.pallas_call(
        body,
        out_shape=jax.ShapeDtypeStruct((m, n), jnp.float32),
        in_specs=[pl.BlockSpec(memory_space=pltpu.VMEM)],
        out_specs=pl.BlockSpec(memory_space=pltpu.VMEM),
        scratch_shapes=[
            pltpu.VMEM((m, n), jnp.bfloat16),
            pltpu.VMEM((N_STAGES, m, n), jnp.bfloat16),
            pltpu.SemaphoreType.DMA,
            pltpu.SemaphoreType.DMA((N_STAGES,)),
        ],
        compiler_params=pltpu.CompilerParams(collective_id=0),
    )(t)


# device time: 22319 ns/iter; 1.2889x vs baseline; 1.2799x over previous
import functools

import jax
import jax.numpy as jnp
from jax import lax
from jax.experimental import pallas as pl
from jax.experimental.pallas import tpu as pltpu

MASKS = (1, 4, 3, 8)
N_STAGES = len(MASKS)
CHUNKS = 4


def kernel(t):
    m, n = t.shape
    rows = m // CHUNKS

    def body(x_ref, out_ref, acc, recv_bufs, send_sems, recv_sems):
        my_i = lax.axis_index("i")

        def sl(c):
            return pl.ds(c * rows, rows)

        def make_rdma(k, c):
            return pltpu.make_async_remote_copy(
                src_ref=acc.at[sl(c), :],
                dst_ref=recv_bufs.at[k, sl(c), :],
                send_sem=send_sems.at[k, c],
                recv_sem=recv_sems.at[k, c],
                device_id=(my_i ^ MASKS[k],),
                device_id_type=pl.DeviceIdType.MESH,
            )

        rdmas = [[make_rdma(k, c) for c in range(CHUNKS)] for k in range(N_STAGES)]

        barrier_sem = pltpu.get_barrier_semaphore()
        for mask in MASKS:
            pl.semaphore_signal(
                barrier_sem, inc=1,
                device_id=(my_i ^ mask,),
                device_id_type=pl.DeviceIdType.MESH,
            )
        pl.semaphore_wait(barrier_sem, N_STAGES)

        for c in range(CHUNKS):
            acc[sl(c), :] = x_ref[sl(c), :].astype(jnp.bfloat16)
            rdmas[0][c].start()

        for k in range(N_STAGES):
            for c in range(CHUNKS):
                rdmas[k][c].wait()
                acc[sl(c), :] = acc[sl(c), :] + recv_bufs[k, sl(c), :]
                if k + 1 < N_STAGES:
                    rdmas[k + 1][c].start()
                else:
                    s = acc[sl(c), :].astype(jnp.float32)
                    r = jnp.maximum(s, 0.0)
                    out_ref[sl(c), :] = jnp.tanh(s) * s * s + r * r * r

        @functools.partial(
            pl.run_scoped, exit_barrier=pltpu.SemaphoreType.REGULAR
        )
        def _(exit_barrier):
            for mask in MASKS:
                pl.semaphore_signal(
                    exit_barrier, inc=1,
                    device_id=(my_i ^ mask,),
                    device_id_type=pl.DeviceIdType.MESH,
                )
            pl.semaphore_wait(exit_barrier, N_STAGES)

    return pl.pallas_call(
        body,
        out_shape=jax.ShapeDtypeStruct((m, n), jnp.float32),
        in_specs=[pl.BlockSpec(memory_space=pltpu.VMEM)],
        out_specs=pl.BlockSpec(memory_space=pltpu.VMEM),
        scratch_shapes=[
            pltpu.VMEM((m, n), jnp.bfloat16),
            pltpu.VMEM((N_STAGES, m, n), jnp.bfloat16),
            pltpu.SemaphoreType.DMA((N_STAGES, CHUNKS)),
            pltpu.SemaphoreType.DMA((N_STAGES, CHUNKS)),
        ],
        compiler_params=pltpu.CompilerParams(collective_id=0),
    )(t)


# device time: 21616 ns/iter; 1.3309x vs baseline; 1.0325x over previous
import functools

import jax
import jax.numpy as jnp
from jax import lax
from jax.experimental import pallas as pl
from jax.experimental.pallas import tpu as pltpu

MASKS = (1, 4, 3, 8)
N_STAGES = len(MASKS)
CHUNKS = 8


def kernel(t):
    m, n = t.shape
    rows = m // CHUNKS

    def body(x_ref, out_ref, acc, recv_bufs, send_sems, recv_sems):
        my_i = lax.axis_index("i")

        def sl(c):
            return pl.ds(c * rows, rows)

        def make_rdma(k, c):
            return pltpu.make_async_remote_copy(
                src_ref=acc.at[sl(c), :],
                dst_ref=recv_bufs.at[k, sl(c), :],
                send_sem=send_sems.at[k, c],
                recv_sem=recv_sems.at[k, c],
                device_id=(my_i ^ MASKS[k],),
                device_id_type=pl.DeviceIdType.MESH,
            )

        rdmas = [[make_rdma(k, c) for c in range(CHUNKS)] for k in range(N_STAGES)]

        barrier_sem = pltpu.get_barrier_semaphore()
        for mask in MASKS:
            pl.semaphore_signal(
                barrier_sem, inc=1,
                device_id=(my_i ^ mask,),
                device_id_type=pl.DeviceIdType.MESH,
            )
        pl.semaphore_wait(barrier_sem, N_STAGES)

        for c in range(CHUNKS):
            acc[sl(c), :] = x_ref[sl(c), :].astype(jnp.bfloat16)
            rdmas[0][c].start()

        for k in range(N_STAGES):
            for c in range(CHUNKS):
                rdmas[k][c].wait()
                acc[sl(c), :] = acc[sl(c), :] + recv_bufs[k, sl(c), :]
                if k + 1 < N_STAGES:
                    rdmas[k + 1][c].start()
                else:
                    s = acc[sl(c), :].astype(jnp.float32)
                    r = jnp.maximum(s, 0.0)
                    out_ref[sl(c), :] = jnp.tanh(s) * s * s + r * r * r

        @functools.partial(
            pl.run_scoped, exit_barrier=pltpu.SemaphoreType.REGULAR
        )
        def _(exit_barrier):
            for mask in MASKS:
                pl.semaphore_signal(
                    exit_barrier, inc=1,
                    device_id=(my_i ^ mask,),
                    device_id_type=pl.DeviceIdType.MESH,
                )
            pl.semaphore_wait(exit_barrier, N_STAGES)

    return pl.pallas_call(
        body,
        out_shape=jax.ShapeDtypeStruct((m, n), jnp.float32),
        in_specs=[pl.BlockSpec(memory_space=pltpu.VMEM)],
        out_specs=pl.BlockSpec(memory_space=pltpu.VMEM),
        scratch_shapes=[
            pltpu.VMEM((m, n), jnp.bfloat16),
            pltpu.VMEM((N_STAGES, m, n), jnp.bfloat16),
            pltpu.SemaphoreType.DMA((N_STAGES, CHUNKS)),
            pltpu.SemaphoreType.DMA((N_STAGES, CHUNKS)),
        ],
        compiler_params=pltpu.CompilerParams(collective_id=0),
    )(t)


# device time: 21569 ns/iter; 1.3338x vs baseline; 1.0022x over previous
import functools

import jax
import jax.numpy as jnp
from jax import lax
from jax.experimental import pallas as pl
from jax.experimental.pallas import tpu as pltpu

STAGES = ((1, 3, 2), (4,), (8,))
N_STAGES = len(STAGES)
SLOT_BASE = (0, 3, 4)
N_SLOTS = 5
ALL_MASKS = tuple(m for st in STAGES for m in st)
CHUNKS = 4


def kernel(t):
    m, n = t.shape
    rows = m // CHUNKS

    def body(x_ref, out_ref, acc, recv_bufs, send_sems, recv_sems):
        my_i = lax.axis_index("i")

        def sl(c):
            return pl.ds(c * rows, rows)

        def make_rdma(k, j, c):
            slot = SLOT_BASE[k] + j
            return pltpu.make_async_remote_copy(
                src_ref=acc.at[sl(c), :],
                dst_ref=recv_bufs.at[slot, sl(c), :],
                send_sem=send_sems.at[slot, c],
                recv_sem=recv_sems.at[slot, c],
                device_id=(my_i ^ STAGES[k][j],),
                device_id_type=pl.DeviceIdType.MESH,
            )

        rdmas = [
            [[make_rdma(k, j, c) for c in range(CHUNKS)]
             for j in range(len(STAGES[k]))]
            for k in range(N_STAGES)
        ]

        barrier_sem = pltpu.get_barrier_semaphore()
        for mask in ALL_MASKS:
            pl.semaphore_signal(
                barrier_sem, inc=1,
                device_id=(my_i ^ mask,),
                device_id_type=pl.DeviceIdType.MESH,
            )
        pl.semaphore_wait(barrier_sem, len(ALL_MASKS))

        for c in range(CHUNKS):
            acc[sl(c), :] = x_ref[sl(c), :].astype(jnp.bfloat16)
            for j in range(len(STAGES[0])):
                rdmas[0][j][c].start()

        for k in range(N_STAGES):
            for c in range(CHUNKS):
                for j in range(len(STAGES[k])):
                    rdmas[k][j][c].wait()
                total = acc[sl(c), :]
                for j in range(len(STAGES[k])):
                    total = total + recv_bufs[SLOT_BASE[k] + j, sl(c), :]
                acc[sl(c), :] = total
                if k + 1 < N_STAGES:
                    for j in range(len(STAGES[k + 1])):
                        rdmas[k + 1][j][c].start()
                else:
                    s = acc[sl(c), :].astype(jnp.float32)
                    r = jnp.maximum(s, 0.0)
                    out_ref[sl(c), :] = jnp.tanh(s) * s * s + r * r * r

        @functools.partial(
            pl.run_scoped, exit_barrier=pltpu.SemaphoreType.REGULAR
        )
        def _(exit_barrier):
            for mask in ALL_MASKS:
                pl.semaphore_signal(
                    exit_barrier, inc=1,
                    device_id=(my_i ^ mask,),
                    device_id_type=pl.DeviceIdType.MESH,
                )
            pl.semaphore_wait(exit_barrier, len(ALL_MASKS))

    return pl.pallas_call(
        body,
        out_shape=jax.ShapeDtypeStruct((m, n), jnp.float32),
        in_specs=[pl.BlockSpec(memory_space=pltpu.VMEM)],
        out_specs=pl.BlockSpec(memory_space=pltpu.VMEM),
        scratch_shapes=[
            pltpu.VMEM((m, n), jnp.bfloat16),
            pltpu.VMEM((N_SLOTS, m, n), jnp.bfloat16),
            pltpu.SemaphoreType.DMA((N_SLOTS, CHUNKS)),
            pltpu.SemaphoreType.DMA((N_SLOTS, CHUNKS)),
        ],
        compiler_params=pltpu.CompilerParams(collective_id=0),
    )(t)
